# baseline (speedup 1.0000x reference)
import functools

import jax
import jax.numpy as jnp
from jax.experimental import pallas as pl
from jax.experimental.pallas import tpu as pltpu
from jax.experimental.pallas import tpu_sc as plsc

_NUM_LAYERS = 2
_S_LEN = 200
_B = 1024
_D = 64
_V = 100000
_N = _S_LEN * _B * _D
_W = 128
_BW = _B // _W
_Q = _B // 2


def _sc_gather(table, idx2):
    mesh = plsc.VectorSubcoreMesh(
        core_axis_name="core", subcore_axis_name="subcore"
    )

    @functools.partial(
        pl.kernel,
        out_type=jax.ShapeDtypeStruct((_S_LEN, _B, _D), jnp.float32),
        mesh=mesh,
        compiler_params=pltpu.CompilerParams(use_tc_tiling_on_sc=False),
    )
    def k(table_hbm, idx_hbm, out_hbm):
        def body(i_vmem, o_vmem):
            pltpu.sync_copy(table_hbm.at[i_vmem.at[0]], o_vmem.at[0])

        pltpu.emit_pipeline(
            body,
            grid=(_S_LEN * _BW,),
            in_specs=[pl.BlockSpec((1, _W), lambda i: (i // _BW, i % _BW))],
            out_specs=[
                pl.BlockSpec((1, _W, _D), lambda i: (i // _BW, i % _BW, 0))
            ],
            core_axis_name=("core", "subcore"),
            dimension_semantics=(pltpu.PARALLEL,),
        )(idx_hbm, out_hbm)

    return k(table, idx2)


_SBLK = 2


def _tc_mean_copy(flat):
    def body(x_ref, mb_ref, e1_ref, e2_ref, acc_ref):
        t = pl.program_id(0)
        x = x_ref[...].reshape(_SBLK * _Q, 2 * _D)
        yt = jnp.swapaxes(x, 0, 1)
        for r in range(_SBLK):
            mb_ref[r, :, : _Q] = yt[: _D, r * _Q : (r + 1) * _Q]
            mb_ref[r, :, _Q :] = yt[_D :, r * _Q : (r + 1) * _Q]

        xs = x.reshape(_SBLK, _Q, 2 * _D).sum(axis=0)

        @pl.when(t == 0)
        def _():
            acc_ref[...] = xs

        @pl.when(t > 0)
        def _():
            acc_ref[...] += xs

        @pl.when(t == _S_LEN // _SBLK - 1)
        def _():
            at = jnp.swapaxes(acc_ref[...], 0, 1) * (1.0 / _S_LEN)
            lo = jnp.broadcast_to(at[None, : _D, :], (_NUM_LAYERS, _D, _Q))
            hi = jnp.broadcast_to(at[None, _D :, :], (_NUM_LAYERS, _D, _Q))
            e1_ref[:, :, : _Q] = lo
            e1_ref[:, :, _Q :] = hi
            e2_ref[:, :, : _Q] = lo
            e2_ref[:, :, _Q :] = hi

    return pl.pallas_call(
        body,
        grid=(_S_LEN // _SBLK,),
        in_specs=[pl.BlockSpec((_SBLK * _B * _D,), lambda t: (t,))],
        out_specs=[
            pl.BlockSpec((_SBLK, _D, _B), lambda t: (t, 0, 0)),
            pl.BlockSpec((_NUM_LAYERS, _D, _B), lambda t: (0, 0, 0)),
            pl.BlockSpec((_NUM_LAYERS, _D, _B), lambda t: (0, 0, 0)),
        ],
        out_shape=[
            jax.ShapeDtypeStruct((_S_LEN, _D, _B), jnp.float32),
            jax.ShapeDtypeStruct((_NUM_LAYERS, _D, _B), jnp.float32),
            jax.ShapeDtypeStruct((_NUM_LAYERS, _D, _B), jnp.float32),
        ],
        scratch_shapes=[pltpu.VMEM((_Q, 2 * _D), jnp.float32)],
    )(flat)


def kernel(src, lengths, table):
    del lengths
    idx = src[..., 0].astype(jnp.int32)
    idx2 = idx.reshape(_S_LEN, 2, _Q).transpose(0, 2, 1).reshape(_S_LEN, _B)
    gathered = _sc_gather(table, idx2)
    flat = gathered.reshape(_N)
    mb_t, ef1_t, ef2_t = _tc_mean_copy(flat)
    memory_bank = jnp.swapaxes(mb_t, 1, 2)
    ef1 = jnp.swapaxes(ef1_t, 1, 2)
    ef2 = jnp.swapaxes(ef2_t, 1, 2)
    return (ef1, ef2, memory_bank)


# speedup vs baseline: 1.2742x; 1.2742x over previous
import functools

import jax
import jax.numpy as jnp
from jax.experimental import pallas as pl
from jax.experimental.pallas import tpu as pltpu
from jax.experimental.pallas import tpu_sc as plsc

_NUM_LAYERS = 2
_S_LEN = 200
_B = 1024
_D = 64
_V = 100000
_N = _S_LEN * _B * _D
_W = 128
_BW = _B // _W
_Q = _B // 2


def _sc_gather(table2, idx2):
    mesh = plsc.VectorSubcoreMesh(
        core_axis_name="core", subcore_axis_name="subcore"
    )

    @functools.partial(
        pl.kernel,
        out_type=jax.ShapeDtypeStruct((_S_LEN, _B, _D), jnp.float32),
        mesh=mesh,
        compiler_params=pltpu.CompilerParams(use_tc_tiling_on_sc=False),
    )
    def k(table_hbm, idx_hbm, out_hbm):
        def body(i_vmem, o_vmem):
            pltpu.sync_copy(table_hbm.at[i_vmem.at[0]], o_vmem.at[0])

        pltpu.emit_pipeline(
            body,
            grid=(_S_LEN * _BW,),
            in_specs=[pl.BlockSpec((1, _W), lambda i: (i // _BW, i % _BW))],
            out_specs=[
                pl.BlockSpec((1, _W, _D), lambda i: (i // _BW, i % _BW, 0))
            ],
            core_axis_name=("core", "subcore"),
            dimension_semantics=(pltpu.PARALLEL,),
        )(idx_hbm, out_hbm)

    return k(table2, idx2)


_SBLK = 8


def _tc_mean_copy(flat):
    def body(x_ref, mb_ref, e1_ref, e2_ref, acc_ref):
        t = pl.program_id(0)
        x = x_ref[...].reshape(_SBLK * _Q, 2 * _D)
        yt = jnp.swapaxes(x, 0, 1)
        for r in range(_SBLK):
            mb_ref[r, :, : _Q] = yt[: _D, r * _Q : (r + 1) * _Q]
            mb_ref[r, :, _Q :] = yt[_D :, r * _Q : (r + 1) * _Q]

        xs = x.reshape(_SBLK, _Q, 2 * _D).sum(axis=0)

        @pl.when(t == 0)
        def _():
            acc_ref[...] = xs

        @pl.when(t > 0)
        def _():
            acc_ref[...] += xs

        @pl.when(t == _S_LEN // _SBLK - 1)
        def _():
            at = jnp.swapaxes(acc_ref[...], 0, 1) * (1.0 / _S_LEN)
            lo = jnp.broadcast_to(at[None, : _D, :], (_NUM_LAYERS, _D, _Q))
            hi = jnp.broadcast_to(at[None, _D :, :], (_NUM_LAYERS, _D, _Q))
            e1_ref[:, :, : _Q] = lo
            e1_ref[:, :, _Q :] = hi
            e2_ref[:, :, : _Q] = lo
            e2_ref[:, :, _Q :] = hi

    return pl.pallas_call(
        body,
        grid=(_S_LEN // _SBLK,),
        in_specs=[pl.BlockSpec((_SBLK * _B * _D,), lambda t: (t,))],
        out_specs=[
            pl.BlockSpec((_SBLK, _D, _B), lambda t: (t, 0, 0)),
            pl.BlockSpec((_NUM_LAYERS, _D, _B), lambda t: (0, 0, 0)),
            pl.BlockSpec((_NUM_LAYERS, _D, _B), lambda t: (0, 0, 0)),
        ],
        out_shape=[
            jax.ShapeDtypeStruct((_S_LEN, _D, _B), jnp.float32),
            jax.ShapeDtypeStruct((_NUM_LAYERS, _D, _B), jnp.float32),
            jax.ShapeDtypeStruct((_NUM_LAYERS, _D, _B), jnp.float32),
        ],
        scratch_shapes=[pltpu.VMEM((_Q, 2 * _D), jnp.float32)],
    )(flat)


def kernel(src, lengths, table):
    del lengths
    idx = src[..., 0].astype(jnp.int32)
    idx2 = (
        (idx * 2).reshape(_S_LEN, 2, _Q).transpose(0, 2, 1).reshape(_S_LEN, _B)
    )
    table_pad = jnp.pad(table, ((0, 0), (0, _D)))
    table2 = table_pad.reshape(2 * _V, _D)
    gathered = _sc_gather(table2, idx2)
    flat = gathered.reshape(_N)
    mb_t, ef1_t, ef2_t = _tc_mean_copy(flat)
    memory_bank = jnp.swapaxes(mb_t, 1, 2)
    ef1 = jnp.swapaxes(ef1_t, 1, 2)
    ef2 = jnp.swapaxes(ef2_t, 1, 2)
    return (ef1, ef2, memory_bank)


# speedup vs baseline: 1.3147x; 1.0318x over previous
import functools

import jax
import jax.numpy as jnp
from jax.experimental import pallas as pl
from jax.experimental.pallas import tpu as pltpu
from jax.experimental.pallas import tpu_sc as plsc

_NUM_LAYERS = 2
_S_LEN = 200
_B = 1024
_D = 64
_V = 100000
_N = _S_LEN * _B * _D
_W = 128
_BW = _B // _W
_Q = _B // 2


def _sc_gather(table2, idx2):
    mesh = plsc.VectorSubcoreMesh(
        core_axis_name="core", subcore_axis_name="subcore"
    )

    @functools.partial(
        pl.kernel,
        out_type=jax.ShapeDtypeStruct((_S_LEN, _B, _D), jnp.float32),
        mesh=mesh,
        compiler_params=pltpu.CompilerParams(use_tc_tiling_on_sc=False),
    )
    def k(table_hbm, idx_hbm, out_hbm):
        def body(i_vmem, o_vmem):
            pltpu.sync_copy(table_hbm.at[i_vmem.at[0]], o_vmem.at[0])

        pltpu.emit_pipeline(
            body,
            grid=(_S_LEN * _BW,),
            in_specs=[pl.BlockSpec((1, _W), lambda i: (i // _BW, i % _BW))],
            out_specs=[
                pl.BlockSpec((1, _W, _D), lambda i: (i // _BW, i % _BW, 0))
            ],
            core_axis_name=("core", "subcore"),
            dimension_semantics=(pltpu.PARALLEL,),
        )(idx_hbm, out_hbm)

    return k(table2, idx2)


_SBLK = 20


def _tc_mean_copy(flat):
    def body(x_ref, mb_ref, e1_ref, e2_ref, acc_ref):
        t = pl.program_id(0)
        x = x_ref[...].reshape(_SBLK * _Q, 2 * _D)
        yt = jnp.swapaxes(x, 0, 1)
        for r in range(_SBLK):
            mb_ref[r, :, : _Q] = yt[: _D, r * _Q : (r + 1) * _Q]
            mb_ref[r, :, _Q :] = yt[_D :, r * _Q : (r + 1) * _Q]

        xs = x.reshape(_SBLK, _Q, 2 * _D).sum(axis=0)

        @pl.when(t == 0)
        def _():
            acc_ref[...] = xs

        @pl.when(t > 0)
        def _():
            acc_ref[...] += xs

        @pl.when(t == _S_LEN // _SBLK - 1)
        def _():
            at = jnp.swapaxes(acc_ref[...], 0, 1) * (1.0 / _S_LEN)
            lo = jnp.broadcast_to(at[None, : _D, :], (_NUM_LAYERS, _D, _Q))
            hi = jnp.broadcast_to(at[None, _D :, :], (_NUM_LAYERS, _D, _Q))
            e1_ref[:, :, : _Q] = lo
            e1_ref[:, :, _Q :] = hi
            e2_ref[:, :, : _Q] = lo
            e2_ref[:, :, _Q :] = hi

    return pl.pallas_call(
        body,
        grid=(_S_LEN // _SBLK,),
        in_specs=[pl.BlockSpec((_SBLK * _B * _D,), lambda t: (t,))],
        out_specs=[
            pl.BlockSpec((_SBLK, _D, _B), lambda t: (t, 0, 0)),
            pl.BlockSpec((_NUM_LAYERS, _D, _B), lambda t: (0, 0, 0)),
            pl.BlockSpec((_NUM_LAYERS, _D, _B), lambda t: (0, 0, 0)),
        ],
        out_shape=[
            jax.ShapeDtypeStruct((_S_LEN, _D, _B), jnp.float32),
            jax.ShapeDtypeStruct((_NUM_LAYERS, _D, _B), jnp.float32),
            jax.ShapeDtypeStruct((_NUM_LAYERS, _D, _B), jnp.float32),
        ],
        scratch_shapes=[pltpu.VMEM((_Q, 2 * _D), jnp.float32)],
    )(flat)


def kernel(src, lengths, table):
    del lengths
    idx = src[..., 0].astype(jnp.int32)
    idx2 = (
        (idx * 2).reshape(_S_LEN, 2, _Q).transpose(0, 2, 1).reshape(_S_LEN, _B)
    )
    table_pad = jnp.pad(table, ((0, 0), (0, _D)))
    table2 = table_pad.reshape(2 * _V, _D)
    gathered = _sc_gather(table2, idx2)
    flat = gathered.reshape(_N)
    mb_t, ef1_t, ef2_t = _tc_mean_copy(flat)
    memory_bank = jnp.swapaxes(mb_t, 1, 2)
    ef1 = jnp.swapaxes(ef1_t, 1, 2)
    ef2 = jnp.swapaxes(ef2_t, 1, 2)
    return (ef1, ef2, memory_bank)


# speedup vs baseline: 1.3156x; 1.0007x over previous
import functools

import jax
import jax.numpy as jnp
from jax.experimental import pallas as pl
from jax.experimental.pallas import tpu as pltpu
from jax.experimental.pallas import tpu_sc as plsc

_NUM_LAYERS = 2
_S_LEN = 200
_B = 1024
_D = 64
_V = 100000
_N = _S_LEN * _B * _D
_W = 128
_BW = _B // _W
_Q = _B // 2


def _sc_gather(table2, idx2):
    mesh = plsc.VectorSubcoreMesh(
        core_axis_name="core", subcore_axis_name="subcore"
    )

    @functools.partial(
        pl.kernel,
        out_type=jax.ShapeDtypeStruct((_S_LEN, _B, _D), jnp.float32),
        mesh=mesh,
        compiler_params=pltpu.CompilerParams(use_tc_tiling_on_sc=False),
    )
    def k(table_hbm, idx_hbm, out_hbm):
        def body(i_vmem, o_vmem):
            pltpu.sync_copy(table_hbm.at[i_vmem.at[0]], o_vmem.at[0])

        pltpu.emit_pipeline(
            body,
            grid=(_S_LEN * _BW,),
            in_specs=[pl.BlockSpec((1, _W), lambda i: (i // _BW, i % _BW))],
            out_specs=[
                pl.BlockSpec((1, _W, _D), lambda i: (i // _BW, i % _BW, 0))
            ],
            core_axis_name=("core", "subcore"),
            dimension_semantics=(pltpu.PARALLEL,),
        )(idx_hbm, out_hbm)

    return k(table2, idx2)


_SBLK = 25


def _tc_mean_copy(flat):
    def body(x_ref, mb_ref, e1_ref, e2_ref, acc_ref):
        t = pl.program_id(0)
        x = x_ref[...].reshape(_SBLK * _Q, 2 * _D)
        yt = jnp.swapaxes(x, 0, 1)
        for r in range(_SBLK):
            mb_ref[r, :, : _Q] = yt[: _D, r * _Q : (r + 1) * _Q]
            mb_ref[r, :, _Q :] = yt[_D :, r * _Q : (r + 1) * _Q]

        xs = x.reshape(_SBLK, _Q, 2 * _D).sum(axis=0)

        @pl.when(t == 0)
        def _():
            acc_ref[...] = xs

        @pl.when(t > 0)
        def _():
            acc_ref[...] += xs

        @pl.when(t == _S_LEN // _SBLK - 1)
        def _():
            at = jnp.swapaxes(acc_ref[...], 0, 1) * (1.0 / _S_LEN)
            lo = jnp.broadcast_to(at[None, : _D, :], (_NUM_LAYERS, _D, _Q))
            hi = jnp.broadcast_to(at[None, _D :, :], (_NUM_LAYERS, _D, _Q))
            e1_ref[:, :, : _Q] = lo
            e1_ref[:, :, _Q :] = hi
            e2_ref[:, :, : _Q] = lo
            e2_ref[:, :, _Q :] = hi

    return pl.pallas_call(
        body,
        grid=(_S_LEN // _SBLK,),
        in_specs=[pl.BlockSpec((_SBLK * _B * _D,), lambda t: (t,))],
        out_specs=[
            pl.BlockSpec((_SBLK, _D, _B), lambda t: (t, 0, 0)),
            pl.BlockSpec((_NUM_LAYERS, _D, _B), lambda t: (0, 0, 0)),
            pl.BlockSpec((_NUM_LAYERS, _D, _B), lambda t: (0, 0, 0)),
        ],
        out_shape=[
            jax.ShapeDtypeStruct((_S_LEN, _D, _B), jnp.float32),
            jax.ShapeDtypeStruct((_NUM_LAYERS, _D, _B), jnp.float32),
            jax.ShapeDtypeStruct((_NUM_LAYERS, _D, _B), jnp.float32),
        ],
        scratch_shapes=[pltpu.VMEM((_Q, 2 * _D), jnp.float32)],
    )(flat)


def kernel(src, lengths, table):
    del lengths
    idx = src[..., 0].astype(jnp.int32)
    idx2 = (
        (idx * 2).reshape(_S_LEN, 2, _Q).transpose(0, 2, 1).reshape(_S_LEN, _B)
    )
    table_pad = jnp.pad(table, ((0, 0), (0, _D)))
    table2 = table_pad.reshape(2 * _V, _D)
    gathered = _sc_gather(table2, idx2)
    flat = gathered.reshape(_N)
    mb_t, ef1_t, ef2_t = _tc_mean_copy(flat)
    memory_bank = jnp.swapaxes(mb_t, 1, 2)
    ef1 = jnp.swapaxes(ef1_t, 1, 2)
    ef2 = jnp.swapaxes(ef2_t, 1, 2)
    return (ef1, ef2, memory_bank)
